# baseline (speedup 1.0000x reference)
import functools

import jax
import jax.numpy as jnp
from jax import lax
from jax.experimental import pallas as pl
from jax.experimental.pallas import tpu as pltpu
from jax.experimental.pallas import tpu_sc as plsc

N = 10000
E = 320000
D = 128
H = 128

NC = 2
NS = 16
NW = NC * NS

CHUNK = 125
EDGES_PER_TILE = E // NW
CHUNKS_PER_TILE = EDGES_PER_TILE // CHUNK
ROWS_PER_TILE = N // NS
K = 128
NLOADS = 78
NEXTRA = (E - NW * NLOADS * K) // K


NB = 3
IB = 6
NG = NLOADS // IB


def _sc_scatter_body(efeat_hbm, dst_hbm, out_hbm,
                     acc_sh, idx_v, buf0, buf1, buf2,
                     sem_l0, sem_l1, sem_l2, sem_s0, sem_s1, sem_s2, sem_i):
    c = lax.axis_index("c")
    s = lax.axis_index("s")
    wid = s * NC + c
    e_base = wid * (NLOADS * K)
    bufs = (buf0, buf1, buf2)
    sem_l = (sem_l0, sem_l1, sem_l2)
    sem_s = (sem_s0, sem_s1, sem_s2)

    def load(j, b):
        pltpu.async_copy(efeat_hbm.at[pl.ds(e_base + j * K, K)],
                         bufs[b], sem_l[b])

    def wait_load(b):
        pltpu.make_async_copy(efeat_hbm.at[pl.ds(e_base, K)],
                              bufs[b], sem_l[b]).wait()

    def idx_ref(g, i):
        return idx_v.at[g % 2, pl.ds(i * K, K)]

    def load_idx(g):
        pltpu.async_copy(dst_hbm.at[pl.ds(e_base + g * IB * K, IB * K)],
                         idx_v.at[g % 2], sem_i)

    def scatter_start(b, g, i):
        pltpu.async_copy(bufs[b], acc_sh.at[idx_ref(g, i)],
                         sem_s[b], add=True)

    def wait_scatter(b, g, i):
        pltpu.make_async_copy(bufs[b], acc_sh.at[idx_ref(g, i)],
                              sem_s[b]).wait()

    load(0, 0)
    load(1, 1)
    pltpu.async_copy(dst_hbm.at[pl.ds(e_base, IB * K)], idx_v.at[0], sem_i)

    zvec = jnp.zeros((16,), jnp.float32)

    def zrow(i, carry):
        r = i // (D // 16)
        q = i % (D // 16)
        buf2[r, pl.ds(q * 16, 16)] = zvec
        return carry

    lax.fori_loop(0, CHUNK * (D // 16), zrow, 0)
    r0 = s * ROWS_PER_TILE
    for p in range(ROWS_PER_TILE // CHUNK):
        pltpu.sync_copy(buf2.at[pl.ds(0, CHUNK)],
                        acc_sh.at[pl.ds(r0 + p * CHUNK, CHUNK)])
    pltpu.make_async_copy(dst_hbm.at[pl.ds(e_base, IB * K)],
                          idx_v.at[0], sem_i).wait()
    plsc.subcore_barrier()

    def outer(g, carry):
        @pl.when(g >= 1)
        def _():
            pltpu.make_async_copy(dst_hbm.at[pl.ds(e_base, IB * K)],
                                  idx_v.at[g % 2], sem_i).wait()

        for i in range(IB):
            j = IB * g + i
            b = i % NB
            t = (b + 2) % NB
            wait_load(b)
            scatter_start(b, g, i)

            @pl.when((j >= 1) & (j < NLOADS - 2))
            def _():
                wait_scatter(t, g, i)

            @pl.when(j < NLOADS - 2)
            def _():
                load(j + 2, t)

            if i == 0:
                @pl.when(g < NG - 1)
                def _():
                    load_idx(g + 1)
        return carry

    lax.fori_loop(0, NG, outer, 0)
    for b in range(NB):
        wait_scatter(b, 0, 0)

    @pl.when(wid < NEXTRA)
    def _():
        e0 = NW * NLOADS * K + wid * K
        pltpu.sync_copy(efeat_hbm.at[pl.ds(e0, K)], buf0)
        pltpu.sync_copy(dst_hbm.at[pl.ds(e0, K)],
                        idx_v.at[0, pl.ds(0, K)])
        pltpu.async_copy(buf0, acc_sh.at[idx_v.at[0, pl.ds(0, K)]],
                         sem_s0, add=True).wait()

    plsc.subcore_barrier()

    o0 = c * N + s * ROWS_PER_TILE
    pltpu.sync_copy(acc_sh.at[pl.ds(s * ROWS_PER_TILE, ROWS_PER_TILE)],
                    out_hbm.at[pl.ds(o0, ROWS_PER_TILE)])


@functools.cache
def _sc_scatter():
    return pl.kernel(
        _sc_scatter_body,
        out_type=jax.ShapeDtypeStruct((NC * N, D), jnp.float32),
        mesh=plsc.VectorSubcoreMesh(core_axis_name="c", subcore_axis_name="s",
                                    num_cores=NC, num_subcores=NS),
        scratch_types=[
            pltpu.VMEM_SHARED((N, D), jnp.float32),
            pltpu.VMEM((2, IB * K), jnp.int32),
            pltpu.VMEM((K, D), jnp.float32),
            pltpu.VMEM((K, D), jnp.float32),
            pltpu.VMEM((K, D), jnp.float32),
            pltpu.SemaphoreType.DMA,
            pltpu.SemaphoreType.DMA,
            pltpu.SemaphoreType.DMA,
            pltpu.SemaphoreType.DMA,
            pltpu.SemaphoreType.DMA,
            pltpu.SemaphoreType.DMA,
            pltpu.SemaphoreType.DMA,
        ],
        compiler_params=pltpu.CompilerParams(use_tc_tiling_on_sc=False),
    )


def _mlp_body(part_ref, nfeat_ref, w1a_ref, w1b_ref, b1_ref, w2_ref, b2_ref,
              gamma_ref, beta_ref, out_ref):
    agg = part_ref[0] + part_ref[1]
    x = nfeat_ref[...]
    h = (jnp.dot(agg, w1a_ref[...], preferred_element_type=jnp.float32)
         + jnp.dot(x, w1b_ref[...], preferred_element_type=jnp.float32)
         + b1_ref[...])
    h = h * jax.nn.sigmoid(h)
    h2 = jnp.dot(h, w2_ref[...], preferred_element_type=jnp.float32) + b2_ref[...]
    mean = jnp.mean(h2, axis=-1, keepdims=True)
    zc = h2 - mean
    var = jnp.mean(zc * zc, axis=-1, keepdims=True)
    out_ref[...] = (zc * lax.rsqrt(var + 1e-5) * gamma_ref[...]
                    + beta_ref[...] + x)


def _mlp(parts3, nfeat, w1a, w1b, b1, w2, b2, gamma, beta, block_rows=5000):
    grid = (N // block_rows,)
    return pl.pallas_call(
        _mlp_body,
        grid=grid,
        in_specs=[
            pl.BlockSpec((NC, block_rows, D), lambda i: (0, i, 0)),
            pl.BlockSpec((block_rows, D), lambda i: (i, 0)),
            pl.BlockSpec((D, H), lambda i: (0, 0)),
            pl.BlockSpec((D, H), lambda i: (0, 0)),
            pl.BlockSpec((1, H), lambda i: (0, 0)),
            pl.BlockSpec((H, D), lambda i: (0, 0)),
            pl.BlockSpec((1, D), lambda i: (0, 0)),
            pl.BlockSpec((1, D), lambda i: (0, 0)),
            pl.BlockSpec((1, D), lambda i: (0, 0)),
        ],
        out_specs=pl.BlockSpec((block_rows, D), lambda i: (i, 0)),
        out_shape=jax.ShapeDtypeStruct((N, D), jnp.float32),
    )(parts3, nfeat, w1a, w1b, b1, w2, b2, gamma, beta)


def kernel(efeat, nfeat, edge_index, W1, b1, W2, b2, gamma, beta):
    parts = _sc_scatter()(efeat, edge_index[1])
    parts3 = parts.reshape(NC, N, D)
    return _mlp(parts3, nfeat, W1[:D], W1[D:], b1.reshape(1, H),
                W2, b2.reshape(1, D), gamma.reshape(1, D), beta.reshape(1, D))


# speedup vs baseline: 1.0744x; 1.0744x over previous
import functools

import jax
import jax.numpy as jnp
from jax import lax
from jax.experimental import pallas as pl
from jax.experimental.pallas import tpu as pltpu
from jax.experimental.pallas import tpu_sc as plsc

N = 10000
E = 320000
D = 128
H = 128

NC = 2
NS = 16
NW = NC * NS

CHUNK = 125
EDGES_PER_TILE = E // NW
CHUNKS_PER_TILE = EDGES_PER_TILE // CHUNK
ROWS_PER_TILE = N // NS
K = 128
NLOADS = 78
NEXTRA = (E - NW * NLOADS * K) // K


NB = 3
IB = 6
NG = NLOADS // IB


def _sc_scatter_body(efeat_hbm, ei_hbm, out_hbm,
                     acc_sh, idx_v, buf0, buf1, buf2,
                     sem_l0, sem_l1, sem_l2, sem_s0, sem_s1, sem_s2, sem_i):
    c = lax.axis_index("c")
    s = lax.axis_index("s")
    wid = s * NC + c
    e_base = wid * (NLOADS * K)
    bufs = (buf0, buf1, buf2)
    sem_l = (sem_l0, sem_l1, sem_l2)
    sem_s = (sem_s0, sem_s1, sem_s2)

    def load(j, b):
        pltpu.async_copy(efeat_hbm.at[pl.ds(e_base + j * K, K)],
                         bufs[b], sem_l[b])

    def wait_load(b):
        pltpu.make_async_copy(efeat_hbm.at[pl.ds(e_base, K)],
                              bufs[b], sem_l[b]).wait()

    def idx_ref(g, i):
        return idx_v.at[g % 2, pl.ds(i * K, K)]

    def load_idx(g):
        pltpu.async_copy(ei_hbm.at[1, pl.ds(e_base + g * IB * K, IB * K)],
                         idx_v.at[g % 2], sem_i)

    def scatter_start(b, g, i):
        pltpu.async_copy(bufs[b], acc_sh.at[idx_ref(g, i)],
                         sem_s[b], add=True)

    def wait_scatter(b, g, i):
        pltpu.make_async_copy(bufs[b], acc_sh.at[idx_ref(g, i)],
                              sem_s[b]).wait()

    load(0, 0)
    load(1, 1)
    pltpu.async_copy(ei_hbm.at[1, pl.ds(e_base, IB * K)], idx_v.at[0], sem_i)

    zvec = jnp.zeros((16,), jnp.float32)

    def zrow(i, carry):
        r = i // (D // 16)
        q = i % (D // 16)
        buf2[r, pl.ds(q * 16, 16)] = zvec
        return carry

    lax.fori_loop(0, CHUNK * (D // 16), zrow, 0)
    r0 = s * ROWS_PER_TILE
    for p in range(ROWS_PER_TILE // CHUNK):
        pltpu.sync_copy(buf2.at[pl.ds(0, CHUNK)],
                        acc_sh.at[pl.ds(r0 + p * CHUNK, CHUNK)])
    pltpu.make_async_copy(ei_hbm.at[1, pl.ds(e_base, IB * K)],
                          idx_v.at[0], sem_i).wait()
    plsc.subcore_barrier()

    def outer(g, carry):
        @pl.when(g >= 1)
        def _():
            pltpu.make_async_copy(ei_hbm.at[1, pl.ds(e_base, IB * K)],
                                  idx_v.at[g % 2], sem_i).wait()

        for i in range(IB):
            j = IB * g + i
            b = i % NB
            t = (b + 2) % NB
            wait_load(b)
            scatter_start(b, g, i)

            @pl.when((j >= 1) & (j < NLOADS - 2))
            def _():
                wait_scatter(t, g, i)

            @pl.when(j < NLOADS - 2)
            def _():
                load(j + 2, t)

            if i == 0:
                @pl.when(g < NG - 1)
                def _():
                    load_idx(g + 1)
        return carry

    lax.fori_loop(0, NG, outer, 0)
    for b in range(NB):
        wait_scatter(b, 0, 0)

    @pl.when(wid < NEXTRA)
    def _():
        e0 = NW * NLOADS * K + wid * K
        pltpu.sync_copy(efeat_hbm.at[pl.ds(e0, K)], buf0)
        pltpu.sync_copy(ei_hbm.at[1, pl.ds(e0, K)],
                        idx_v.at[0, pl.ds(0, K)])
        pltpu.async_copy(buf0, acc_sh.at[idx_v.at[0, pl.ds(0, K)]],
                         sem_s0, add=True).wait()

    plsc.subcore_barrier()

    o0 = c * N + s * ROWS_PER_TILE
    pltpu.sync_copy(acc_sh.at[pl.ds(s * ROWS_PER_TILE, ROWS_PER_TILE)],
                    out_hbm.at[pl.ds(o0, ROWS_PER_TILE)])


@functools.cache
def _sc_scatter():
    return pl.kernel(
        _sc_scatter_body,
        out_type=jax.ShapeDtypeStruct((NC * N, D), jnp.float32),
        mesh=plsc.VectorSubcoreMesh(core_axis_name="c", subcore_axis_name="s",
                                    num_cores=NC, num_subcores=NS),
        scratch_types=[
            pltpu.VMEM_SHARED((N, D), jnp.float32),
            pltpu.VMEM((2, IB * K), jnp.int32),
            pltpu.VMEM((K, D), jnp.float32),
            pltpu.VMEM((K, D), jnp.float32),
            pltpu.VMEM((K, D), jnp.float32),
            pltpu.SemaphoreType.DMA,
            pltpu.SemaphoreType.DMA,
            pltpu.SemaphoreType.DMA,
            pltpu.SemaphoreType.DMA,
            pltpu.SemaphoreType.DMA,
            pltpu.SemaphoreType.DMA,
            pltpu.SemaphoreType.DMA,
        ],
        compiler_params=pltpu.CompilerParams(use_tc_tiling_on_sc=False),
    )


def _mlp_body(part_ref, nfeat_ref, w1a_ref, w1b_ref, b1_ref, w2_ref, b2_ref,
              gamma_ref, beta_ref, out_ref):
    agg = part_ref[0] + part_ref[1]
    x = nfeat_ref[...]
    h = (jnp.dot(agg, w1a_ref[...], preferred_element_type=jnp.float32)
         + jnp.dot(x, w1b_ref[...], preferred_element_type=jnp.float32)
         + b1_ref[...])
    h = h * jax.nn.sigmoid(h)
    h2 = jnp.dot(h, w2_ref[...], preferred_element_type=jnp.float32) + b2_ref[...]
    mean = jnp.mean(h2, axis=-1, keepdims=True)
    zc = h2 - mean
    var = jnp.mean(zc * zc, axis=-1, keepdims=True)
    out_ref[...] = (zc * lax.rsqrt(var + 1e-5) * gamma_ref[...]
                    + beta_ref[...] + x)


def _mlp(parts3, nfeat, w1a, w1b, b1, w2, b2, gamma, beta, block_rows=5000):
    grid = (N // block_rows,)
    return pl.pallas_call(
        _mlp_body,
        grid=grid,
        in_specs=[
            pl.BlockSpec((NC, block_rows, D), lambda i: (0, i, 0)),
            pl.BlockSpec((block_rows, D), lambda i: (i, 0)),
            pl.BlockSpec((D, H), lambda i: (0, 0)),
            pl.BlockSpec((D, H), lambda i: (0, 0)),
            pl.BlockSpec((1, H), lambda i: (0, 0)),
            pl.BlockSpec((H, D), lambda i: (0, 0)),
            pl.BlockSpec((1, D), lambda i: (0, 0)),
            pl.BlockSpec((1, D), lambda i: (0, 0)),
            pl.BlockSpec((1, D), lambda i: (0, 0)),
        ],
        out_specs=pl.BlockSpec((block_rows, D), lambda i: (i, 0)),
        out_shape=jax.ShapeDtypeStruct((N, D), jnp.float32),
    )(parts3, nfeat, w1a, w1b, b1, w2, b2, gamma, beta)


def kernel(efeat, nfeat, edge_index, W1, b1, W2, b2, gamma, beta):
    parts = _sc_scatter()(efeat, edge_index)
    parts3 = parts.reshape(NC, N, D)
    return _mlp(parts3, nfeat, W1[:D], W1[D:], b1.reshape(1, H),
                W2, b2.reshape(1, D), gamma.reshape(1, D), beta.reshape(1, D))


# speedup vs baseline: 1.0775x; 1.0029x over previous
import functools

import jax
import jax.numpy as jnp
from jax import lax
from jax.experimental import pallas as pl
from jax.experimental.pallas import tpu as pltpu
from jax.experimental.pallas import tpu_sc as plsc

N = 10000
E = 320000
D = 128
H = 128

NC = 2
NS = 16
NW = NC * NS

CHUNK = 125
ROWS_PER_TILE = N // NS
K = 128
NLOADS = 78
NEXTRA = (E - NW * NLOADS * K) // K
NB = 3
IB = 6
NG = NLOADS // IB


def _sc_scatter_body(efeat_hbm, ei_hbm, out_hbm,
                     acc_sh, idx_v, buf0, buf1, buf2,
                     sem_l0, sem_l1, sem_l2, sem_s0, sem_s1, sem_s2, sem_i):
    c = lax.axis_index("c")
    s = lax.axis_index("s")
    wid = s * NC + c
    e_base = wid * (NLOADS * K)
    bufs = (buf0, buf1, buf2)
    sem_l = (sem_l0, sem_l1, sem_l2)
    sem_s = (sem_s0, sem_s1, sem_s2)

    def load(j, b):
        pltpu.async_copy(efeat_hbm.at[pl.ds(e_base + j * K, K)],
                         bufs[b], sem_l[b])

    def wait_load(b):
        pltpu.make_async_copy(efeat_hbm.at[pl.ds(e_base, K)],
                              bufs[b], sem_l[b]).wait()

    def idx_ref(g, i):
        return idx_v.at[g % 2, pl.ds(i * K, K)]

    def load_idx(g):
        pltpu.async_copy(ei_hbm.at[1, pl.ds(e_base + g * IB * K, IB * K)],
                         idx_v.at[g % 2], sem_i)

    def scatter_start(b, g, i):
        pltpu.async_copy(bufs[b], acc_sh.at[idx_ref(g, i)],
                         sem_s[b], add=True)

    def wait_scatter(b, g, i):
        pltpu.make_async_copy(bufs[b], acc_sh.at[idx_ref(g, i)],
                              sem_s[b]).wait()

    load(0, 0)
    load(1, 1)
    pltpu.async_copy(ei_hbm.at[1, pl.ds(e_base, IB * K)], idx_v.at[0], sem_i)

    zvec = jnp.zeros((16,), jnp.float32)

    def zrow(i, carry):
        r = i // (D // 16)
        q = i % (D // 16)
        buf2[r, pl.ds(q * 16, 16)] = zvec
        return carry

    lax.fori_loop(0, CHUNK * (D // 16), zrow, 0)
    r0 = s * ROWS_PER_TILE
    for p in range(ROWS_PER_TILE // CHUNK):
        pltpu.sync_copy(buf2.at[pl.ds(0, CHUNK)],
                        acc_sh.at[pl.ds(r0 + p * CHUNK, CHUNK)])
    pltpu.make_async_copy(ei_hbm.at[1, pl.ds(e_base, IB * K)],
                          idx_v.at[0], sem_i).wait()
    plsc.subcore_barrier()

    def outer(g, carry):
        @pl.when(g >= 1)
        def _():
            pltpu.make_async_copy(ei_hbm.at[1, pl.ds(e_base, IB * K)],
                                  idx_v.at[g % 2], sem_i).wait()

        for i in range(IB):
            j = IB * g + i
            b = i % NB
            t = (b + 2) % NB
            wait_load(b)
            scatter_start(b, g, i)

            @pl.when((j >= 1) & (j < NLOADS - 2))
            def _():
                wait_scatter(t, g, i)

            @pl.when(j < NLOADS - 2)
            def _():
                load(j + 2, t)

            if i == 0:
                @pl.when(g < NG - 1)
                def _():
                    load_idx(g + 1)
        return carry

    lax.fori_loop(0, NG, outer, 0)
    for b in range(NB):
        wait_scatter(b, 0, 0)

    @pl.when(wid < NEXTRA)
    def _():
        e0 = NW * NLOADS * K + wid * K
        pltpu.sync_copy(efeat_hbm.at[pl.ds(e0, K)], buf0)
        pltpu.sync_copy(ei_hbm.at[1, pl.ds(e0, K)],
                        idx_v.at[0, pl.ds(0, K)])
        pltpu.async_copy(buf0, acc_sh.at[idx_v.at[0, pl.ds(0, K)]],
                         sem_s0, add=True).wait()

    plsc.subcore_barrier()

    o0 = c * N + s * ROWS_PER_TILE
    pltpu.sync_copy(acc_sh.at[pl.ds(s * ROWS_PER_TILE, ROWS_PER_TILE)],
                    out_hbm.at[pl.ds(o0, ROWS_PER_TILE)])


@functools.cache
def _sc_scatter():
    return pl.kernel(
        _sc_scatter_body,
        out_type=jax.ShapeDtypeStruct((NC * N, D), jnp.float32),
        mesh=plsc.VectorSubcoreMesh(core_axis_name="c", subcore_axis_name="s",
                                    num_cores=NC, num_subcores=NS),
        scratch_types=[
            pltpu.VMEM_SHARED((N, D), jnp.float32),
            pltpu.VMEM((2, IB * K), jnp.int32),
            pltpu.VMEM((K, D), jnp.float32),
            pltpu.VMEM((K, D), jnp.float32),
            pltpu.VMEM((K, D), jnp.float32),
            pltpu.SemaphoreType.DMA,
            pltpu.SemaphoreType.DMA,
            pltpu.SemaphoreType.DMA,
            pltpu.SemaphoreType.DMA,
            pltpu.SemaphoreType.DMA,
            pltpu.SemaphoreType.DMA,
            pltpu.SemaphoreType.DMA,
        ],
        compiler_params=pltpu.CompilerParams(use_tc_tiling_on_sc=False),
    )


def _mlp_body(part_ref, nfeat_ref, w1a_ref, w1b_ref, b1_ref, w2_ref, b2_ref,
              gamma_ref, beta_ref, out_ref):
    agg = part_ref[0] + part_ref[1]
    x = nfeat_ref[...]
    h = (jnp.dot(agg, w1a_ref[...], preferred_element_type=jnp.float32)
         + jnp.dot(x, w1b_ref[...], preferred_element_type=jnp.float32)
         + b1_ref[...])
    h = h * jax.nn.sigmoid(h)
    h2 = jnp.dot(h, w2_ref[...], preferred_element_type=jnp.float32) + b2_ref[...]
    mean = jnp.mean(h2, axis=-1, keepdims=True)
    zc = h2 - mean
    var = jnp.mean(zc * zc, axis=-1, keepdims=True)
    out_ref[...] = (zc * lax.rsqrt(var + 1e-5) * gamma_ref[...]
                    + beta_ref[...] + x)


def _mlp(parts3, nfeat, w1a, w1b, b1, w2, b2, gamma, beta, block_rows=5000):
    grid = (N // block_rows,)
    return pl.pallas_call(
        _mlp_body,
        grid=grid,
        in_specs=[
            pl.BlockSpec((NC, block_rows, D), lambda i: (0, i, 0)),
            pl.BlockSpec((block_rows, D), lambda i: (i, 0)),
            pl.BlockSpec((D, H), lambda i: (0, 0)),
            pl.BlockSpec((D, H), lambda i: (0, 0)),
            pl.BlockSpec((1, H), lambda i: (0, 0)),
            pl.BlockSpec((H, D), lambda i: (0, 0)),
            pl.BlockSpec((1, D), lambda i: (0, 0)),
            pl.BlockSpec((1, D), lambda i: (0, 0)),
            pl.BlockSpec((1, D), lambda i: (0, 0)),
        ],
        out_specs=pl.BlockSpec((block_rows, D), lambda i: (i, 0)),
        out_shape=jax.ShapeDtypeStruct((N, D), jnp.float32),
    )(parts3, nfeat, w1a, w1b, b1, w2, b2, gamma, beta)


def kernel(efeat, nfeat, edge_index, W1, b1, W2, b2, gamma, beta):
    parts = _sc_scatter()(efeat, edge_index)
    parts3 = parts.reshape(NC, N, D)
    return _mlp(parts3, nfeat, W1[:D], W1[D:], b1.reshape(1, H),
                W2, b2.reshape(1, D), gamma.reshape(1, D), beta.reshape(1, D))
